# Initial kernel scaffold; baseline (speedup 1.0000x reference)
import functools

import jax
import jax.numpy as jnp
from jax import lax
from jax.experimental import pallas as pl
from jax.experimental.pallas import tpu as pltpu
from jax.experimental.pallas import tpu_sc as plsc

H = 4
C = 128
ED = 16
ACCW = 160
NB = 80
INV_SQRT_C = 1.0 / float(C) ** 0.5


def _pre_body(x_ref, wq_ref, wk_ref, wv_ref, we_ref, ws_ref, bq_ref, bk_ref,
              bv_ref, bs_ref, q_ref, k_ref, v_ref, qe_ref, skip_ref):
    xb = x_ref[...]
    dn = (((1,), (1,)), ((), ()))
    q = lax.dot_general(xb, wq_ref[...], dn,
                        preferred_element_type=jnp.float32) + bq_ref[...]
    k = lax.dot_general(xb, wk_ref[...], dn,
                        preferred_element_type=jnp.float32) + bk_ref[...]
    v = lax.dot_general(xb, wv_ref[...], dn,
                        preferred_element_type=jnp.float32) + bv_ref[...]
    q_ref[...] = q
    k_ref[...] = k
    v_ref[...] = v
    skip_ref[...] = lax.dot_general(
        xb, ws_ref[...], dn, preferred_element_type=jnp.float32) + bs_ref[...]
    we = we_ref[...]
    for h in range(H):
        qe_ref[:, h * ED:(h + 1) * ED] = lax.dot_general(
            q[:, h * C:(h + 1) * C], we[h * C:(h + 1) * C, :],
            (((1,), (0,)), ((), ())), preferred_element_type=jnp.float32)


def _pre(x, p):
    n = x.shape[0]
    bn = 1000
    grid = (n // bn,)
    full = lambda s: pl.BlockSpec(s, lambda i: (0,) * len(s))
    row = lambda w: pl.BlockSpec((bn, w), lambda i: (i, 0))
    return pl.pallas_call(
        _pre_body,
        grid=grid,
        in_specs=[row(x.shape[1])] + [full(p[k].shape) for k in
                                      ("Wq", "Wk", "Wv", "We", "Wskip")]
        + [full((1, H * C))] * 3 + [full((1, C))],
        out_specs=[row(H * C), row(H * C), row(H * C), row(H * ED), row(C)],
        out_shape=[
            jax.ShapeDtypeStruct((n, H * C), jnp.float32),
            jax.ShapeDtypeStruct((n, H * C), jnp.float32),
            jax.ShapeDtypeStruct((n, H * C), jnp.float32),
            jax.ShapeDtypeStruct((n, H * ED), jnp.float32),
            jax.ShapeDtypeStruct((n, C), jnp.float32),
        ],
    )(x, p["Wq"], p["Wk"], p["Wv"], p["We"], p["Wskip"],
      p["bq"].reshape(1, -1), p["bk"].reshape(1, -1), p["bv"].reshape(1, -1),
      p["bskip"].reshape(1, -1))


def _edge_body(nnodes, ept, q_hbm, k_hbm, v_hbm, qe_hbm, ea_hbm, src_hbm,
               dst_hbm, acc_hbm, srcall, dstall, qidx, kidx, dstn, qrows,
               krows, vrows, qerows, earows, exbuf, obuf, zbuf, shared, sem):
    core = lax.axis_index("c")
    sub = lax.axis_index("s")
    ebase = sub * ept
    npt = nnodes // 16
    nbase = sub * npt
    nbatch = ept // NB
    pltpu.sync_copy(src_hbm.at[pl.ds(ebase, ept)], srcall)
    pltpu.sync_copy(dst_hbm.at[pl.ds(ebase, ept)], dstall)
    zeros16 = jnp.zeros((16,), jnp.float32)
    onehot = jnp.where(lax.broadcasted_iota(jnp.int32, (16,), 0) == 0,
                       1.0, 0.0).astype(jnp.float32)

    def zero_zbuf(i, carry):
        for j in range(ACCW // 16):
            zbuf[i, pl.ds(j * 16, 16)] = zeros16
        return carry
    lax.fori_loop(0, zbuf.shape[0], zero_zbuf, 0)

    for hl in range(2):
        h = core * 2 + hl
        zr = zbuf.shape[0]
        for z in range(npt // zr):
            pltpu.sync_copy(zbuf, shared.at[pl.ds(nbase + z * zr, zr)])
        plsc.subcore_barrier()

        def batch_body(b, carry):
            off = b * NB
            for i in range(NB // 16):
                s16 = srcall[pl.ds(off + i * 16, 16)]
                d16 = dstall[pl.ds(off + i * 16, 16)]
                kidx[pl.ds(i * 16, 16)] = s16 * H + h
                qidx[pl.ds(i * 16, 16)] = d16 * H + h
                dstn[pl.ds(i * 16, 16)] = d16
            cps = (pltpu.async_copy(q_hbm.at[qidx], qrows, sem),
                   pltpu.async_copy(k_hbm.at[kidx], krows, sem),
                   pltpu.async_copy(v_hbm.at[kidx], vrows, sem),
                   pltpu.async_copy(qe_hbm.at[qidx], qerows, sem),
                   pltpu.async_copy(ea_hbm.at[pl.ds(ebase + off, NB)],
                                    earows, sem))
            for cp in cps:
                cp.wait()

            def alpha_body(i, c2):
                acc = qrows[i, pl.ds(0, 16)] * krows[i, pl.ds(0, 16)]
                for j in range(1, C // 16):
                    acc += qrows[i, pl.ds(j * 16, 16)] * krows[i, pl.ds(j * 16, 16)]
                acc2 = qerows[i, :] * earows[i, :]
                exbuf[i] = (jnp.sum(acc) + jnp.sum(acc2)) * INV_SQRT_C
                return c2
            lax.fori_loop(0, NB, alpha_body, 0)
            for i in range(NB // 16):
                exbuf[pl.ds(i * 16, 16)] = jnp.exp(exbuf[pl.ds(i * 16, 16)])

            def scale_body(i, c2):
                ex = exbuf[i]
                for j in range(C // 16):
                    obuf[i, pl.ds(j * 16, 16)] = vrows[i, pl.ds(j * 16, 16)] * ex
                obuf[i, pl.ds(C, 16)] = earows[i, :] * ex
                obuf[i, pl.ds(C + 16, 16)] = onehot * ex
                return c2
            lax.fori_loop(0, NB, scale_body, 0)
            pltpu.sync_copy(obuf, shared.at[dstn], add=True)
            return carry
        lax.fori_loop(0, nbatch, batch_body, 0)
        plsc.subcore_barrier()
        pltpu.sync_copy(shared.at[pl.ds(nbase, npt)],
                        acc_hbm.at[h, pl.ds(nbase, npt)])
        plsc.subcore_barrier()


def _edge_pass(q, k, v, qe, ea, src, dst):
    n = q.shape[0]
    e = src.shape[0]
    ept = e // 16
    mesh = plsc.VectorSubcoreMesh(core_axis_name="c", subcore_axis_name="s")
    kern = pl.kernel(
        functools.partial(_edge_body, n, ept),
        out_type=jax.ShapeDtypeStruct((H, n, ACCW), jnp.float32),
        mesh=mesh,
        scratch_types=[
            pltpu.VMEM((ept,), jnp.int32),
            pltpu.VMEM((ept,), jnp.int32),
            pltpu.VMEM((NB,), jnp.int32),
            pltpu.VMEM((NB,), jnp.int32),
            pltpu.VMEM((NB,), jnp.int32),
            pltpu.VMEM((NB, C), jnp.float32),
            pltpu.VMEM((NB, C), jnp.float32),
            pltpu.VMEM((NB, C), jnp.float32),
            pltpu.VMEM((NB, ED), jnp.float32),
            pltpu.VMEM((NB, ED), jnp.float32),
            pltpu.VMEM((NB,), jnp.float32),
            pltpu.VMEM((NB, ACCW), jnp.float32),
            pltpu.VMEM((125, ACCW), jnp.float32),
            pltpu.VMEM_SHARED((n, ACCW), jnp.float32),
            pltpu.SemaphoreType.DMA,
        ],
    )
    qr = q.reshape(n * H, C)
    kr = k.reshape(n * H, C)
    vr = v.reshape(n * H, C)
    qer = qe.reshape(n * H, ED)
    return kern(qr, kr, vr, qer, ea, src, dst)


def _combine_body(acc_ref, skip_ref, we_ref, g_ref, b_ref, wo_ref, bo_ref,
                  o_ref, *, relu, proj):
    a = acc_ref[...]
    we = we_ref[...]
    tot = None
    for h in range(H):
        num = a[h, :, 0:C]
        s = a[h, :, C:C + ED]
        den = a[h, :, C + ED:C + ED + 1]
        numh = num + lax.dot_general(s, we[h * C:(h + 1) * C, :],
                                     (((1,), (1,)), ((), ())),
                                     preferred_element_type=jnp.float32)
        outh = numh / (den + 1e-16)
        tot = outh if tot is None else tot + outh
    m = tot * (1.0 / H) + skip_ref[...]
    mu = jnp.mean(m, axis=-1, keepdims=True)
    var = jnp.mean((m - mu) ** 2, axis=-1, keepdims=True)
    y = (m - mu) / jnp.sqrt(var + 1e-5) * g_ref[...] + b_ref[...]
    if relu:
        y = jnp.maximum(y, 0.0)
    if proj:
        y = lax.dot_general(y, wo_ref[...], (((1,), (1,)), ((), ())),
                            preferred_element_type=jnp.float32) + bo_ref[...]
    o_ref[...] = y


def _combine(acc, skip, we, g, b, wo, bo, relu, proj):
    n = skip.shape[0]
    bn = 1000
    out_w = wo.shape[0] if proj else C
    full = lambda s: pl.BlockSpec(s, lambda i: (0,) * len(s))
    return pl.pallas_call(
        functools.partial(_combine_body, relu=relu, proj=proj),
        grid=(n // bn,),
        in_specs=[pl.BlockSpec((H, bn, ACCW), lambda i: (0, i, 0)),
                  pl.BlockSpec((bn, C), lambda i: (i, 0)),
                  full(we.shape), full((1, C)), full((1, C)),
                  full(wo.shape), full((1, wo.shape[0]))],
        out_specs=pl.BlockSpec((bn, out_w), lambda i: (i, 0)),
        out_shape=jax.ShapeDtypeStruct((n, out_w), jnp.float32),
    )(acc, skip, we, g.reshape(1, -1), b.reshape(1, -1), wo,
      bo.reshape(1, -1))


def kernel(x, edge_index, edge_attr, params):
    src = edge_index[0].astype(jnp.int32)
    dst = edge_index[1].astype(jnp.int32)
    g, b = params["ln_g"], params["ln_b"]

    p = params["c1"]
    q, k, v, qe, skip = _pre(x, p)
    acc = _edge_pass(q, k, v, qe, edge_attr, src, dst)
    h1 = _combine(acc, skip, p["We"], g, b, params["Wout"], params["bout"],
                  relu=True, proj=False)

    p = params["c2"]
    q, k, v, qe, skip = _pre(h1, p)
    acc = _edge_pass(q, k, v, qe, edge_attr, src, dst)
    out = _combine(acc, skip, p["We"], g, b, params["Wout"], params["bout"],
                   relu=False, proj=True)
    return out


# speedup vs baseline: 6.8677x; 6.8677x over previous
import functools

import jax
import jax.numpy as jnp
from jax import lax
from jax.experimental import pallas as pl
from jax.experimental.pallas import tpu as pltpu
from jax.experimental.pallas import tpu_sc as plsc

H = 4
C = 128
ED = 16
QXW = 256
NB = 32
INV_SQRT_C = 1.0 / float(C) ** 0.5


def _pre_body(x_ref, wq_ref, wk_ref, wv_ref, we_ref, ws_ref, bq_ref, bk_ref,
              bv_ref, bs_ref, qx_ref, k_ref, v_ref, skip_ref):
    xb = x_ref[...]
    dn = (((1,), (1,)), ((), ()))
    q = lax.dot_general(xb, wq_ref[...], dn,
                        preferred_element_type=jnp.float32) + bq_ref[...]
    k_ref[...] = lax.dot_general(xb, wk_ref[...], dn,
                                 preferred_element_type=jnp.float32) + bk_ref[...]
    v_ref[...] = lax.dot_general(xb, wv_ref[...], dn,
                                 preferred_element_type=jnp.float32) + bv_ref[...]
    skip_ref[...] = lax.dot_general(
        xb, ws_ref[...], dn, preferred_element_type=jnp.float32) + bs_ref[...]
    we = we_ref[...]
    zpad = jnp.zeros((q.shape[0], QXW - C - ED), jnp.float32)
    for h in range(H):
        qh = q[:, h * C:(h + 1) * C]
        qe = lax.dot_general(qh, we[h * C:(h + 1) * C, :],
                             (((1,), (0,)), ((), ())),
                             preferred_element_type=jnp.float32)
        qx_ref[:, h * QXW:h * QXW + C] = qh
        qx_ref[:, h * QXW + C:h * QXW + C + ED] = qe
        qx_ref[:, h * QXW + C + ED:(h + 1) * QXW] = zpad


def _pre(x, p):
    n = x.shape[0]
    bn = 1000
    full = lambda s: pl.BlockSpec(s, lambda i: (0,) * len(s))
    row = lambda w: pl.BlockSpec((bn, w), lambda i: (i, 0))
    return pl.pallas_call(
        _pre_body,
        grid=(n // bn,),
        in_specs=[row(x.shape[1])] + [full(p[k].shape) for k in
                                      ("Wq", "Wk", "Wv", "We", "Wskip")]
        + [full((1, H * C))] * 3 + [full((1, C))],
        out_specs=[row(H * QXW), row(H * C), row(H * C), row(C)],
        out_shape=[
            jax.ShapeDtypeStruct((n, H * QXW), jnp.float32),
            jax.ShapeDtypeStruct((n, H * C), jnp.float32),
            jax.ShapeDtypeStruct((n, H * C), jnp.float32),
            jax.ShapeDtypeStruct((n, C), jnp.float32),
        ],
    )(x, p["Wq"], p["Wk"], p["Wv"], p["We"], p["Wskip"],
      p["bq"].reshape(1, -1), p["bk"].reshape(1, -1), p["bv"].reshape(1, -1),
      p["bskip"].reshape(1, -1))


def _edge_body(n_pad, ept, e_all, qx_hbm, k_hbm, v_hbm, ea_hbm, qi_hbm,
               ki_hbm, dst_hbm, num_hbm, aux_hbm, ex_hbm, dstb, qidx, kidx,
               qxrows, krows, vrows, earows, exbuf, ex2buf, obuf, zbuf,
               shared, sem):
    core = lax.axis_index("c")
    sub = lax.axis_index("s")
    ebase = sub * ept
    npt = n_pad // 16
    nbase = sub * npt
    nbatch = ept // NB
    zr = zbuf.shape[0]
    zeros16 = jnp.zeros((16,), jnp.float32)
    lane = lax.broadcasted_iota(jnp.int32, (16,), 0)
    onehot = jnp.where(lane == 0, 1.0, 0.0).astype(jnp.float32)
    gd = lax.GatherDimensionNumbers(
        offset_dims=(), collapsed_slice_dims=(0,), start_index_map=(0,))

    def lanesum(v):
        for sh in (8, 4, 2, 1):
            p = lax.gather(v, (lane ^ sh)[:, None], gd, slice_sizes=(1,),
                           mode=lax.GatherScatterMode.PROMISE_IN_BOUNDS)
            v = v + p
        return v

    def zero_zbuf(i, carry):
        for j in range(zbuf.shape[1] // 16):
            zbuf[i, pl.ds(j * 16, 16)] = zeros16
        return carry
    lax.fori_loop(0, zr, zero_zbuf, 0)

    def zero_shared():
        for z in range(npt // zr):
            pltpu.sync_copy(zbuf, shared.at[pl.ds(nbase + z * zr, zr)])


    for hl in range(2):
        h = core * 2 + hl
        zero_shared()
        plsc.subcore_barrier()

        def batch_body(b, carry):
            off = b * NB
            pltpu.sync_copy(qi_hbm.at[pl.ds(h * e_all + ebase + off, NB)],
                            qidx)
            pltpu.sync_copy(ki_hbm.at[pl.ds(h * e_all + ebase + off, NB)],
                            kidx)
            pltpu.sync_copy(dst_hbm.at[pl.ds(ebase + off, NB)], dstb)
            cps = (pltpu.async_copy(qx_hbm.at[qidx], qxrows, sem),
                   pltpu.async_copy(k_hbm.at[kidx], krows, sem),
                   pltpu.async_copy(v_hbm.at[kidx], vrows, sem),
                   pltpu.async_copy(ea_hbm.at[pl.ds(ebase + off, NB)],
                                    earows, sem))
            for cp in cps:
                cp.wait()

            def alpha_body(g, c2):
                base16 = g * 16
                avec = zeros16
                for i16 in range(16):
                    i = base16 + i16
                    acc = qxrows[i, pl.ds(0, 16)] * krows[i, pl.ds(0, 16)]
                    for j in range(1, C // 16):
                        acc += (qxrows[i, pl.ds(j * 16, 16)]
                                * krows[i, pl.ds(j * 16, 16)])
                    acc = acc + qxrows[i, pl.ds(C, 16)] * earows[i, :]
                    a = lanesum(acc)[0]
                    avec = jnp.where(lane == i16, a, avec)
                exbuf[pl.ds(base16, 16)] = jnp.exp(avec * INV_SQRT_C)
                return c2
            lax.fori_loop(0, NB // 16, alpha_body, 0)
            pltpu.sync_copy(exbuf,
                            ex_hbm.at[pl.ds(h * e_all + ebase + off, NB)])

            def scale_body(g, c2):
                base16 = g * 16
                exvec = exbuf[pl.ds(base16, 16)]
                for i16 in range(16):
                    i = base16 + i16
                    ex = exvec[i16]
                    for j in range(C // 16):
                        obuf[i, pl.ds(j * 16, 16)] = (
                            vrows[i, pl.ds(j * 16, 16)] * ex)
                return c2
            lax.fori_loop(0, NB // 16, scale_body, 0)
            pltpu.sync_copy(obuf, shared.at[dstb], add=True)
            return carry
        lax.fori_loop(0, nbatch, batch_body, 0)
        plsc.subcore_barrier()
        pltpu.sync_copy(shared.at[pl.ds(nbase, npt)],
                        num_hbm.at[h, pl.ds(nbase, npt)])
        plsc.subcore_barrier()

    zero_shared()

    def zero_obuf(i, carry):
        for j in range(C // 16):
            obuf[i, pl.ds(j * 16, 16)] = zeros16
        return carry
    lax.fori_loop(0, NB, zero_obuf, 0)
    plsc.subcore_barrier()

    def aux_body(b, carry):
        off = b * NB
        pltpu.sync_copy(dst_hbm.at[pl.ds(ebase + off, NB)], dstb)
        h0 = core * 2
        cps = (pltpu.async_copy(
                   ex_hbm.at[pl.ds(h0 * e_all + ebase + off, NB)], exbuf, sem),
               pltpu.async_copy(
                   ex_hbm.at[pl.ds((h0 + 1) * e_all + ebase + off, NB)],
                   ex2buf, sem),
               pltpu.async_copy(ea_hbm.at[pl.ds(ebase + off, NB)],
                                earows, sem))
        for cp in cps:
            cp.wait()

        def aux_scale(g, c2):
            base16 = g * 16
            exvec0 = exbuf[pl.ds(base16, 16)]
            exvec1 = ex2buf[pl.ds(base16, 16)]
            for i16 in range(16):
                i = base16 + i16
                ex0 = exvec0[i16]
                ex1 = exvec1[i16]
                obuf[i, pl.ds(0, 16)] = earows[i, :] * ex0
                obuf[i, pl.ds(16, 16)] = onehot * ex0
                obuf[i, pl.ds(32, 16)] = earows[i, :] * ex1
                obuf[i, pl.ds(48, 16)] = onehot * ex1
            return c2
        lax.fori_loop(0, NB // 16, aux_scale, 0)
        pltpu.sync_copy(obuf, shared.at[dstb], add=True)
        return carry
    lax.fori_loop(0, nbatch, aux_body, 0)
    plsc.subcore_barrier()
    pltpu.sync_copy(shared.at[pl.ds(nbase, npt)],
                    aux_hbm.at[core, pl.ds(nbase, npt)])


def _edge_pass(qx, k, v, ea, src, dst):
    n = k.shape[0]
    e = src.shape[0]
    ept = e // 16
    n_pad = ((n + 16 * 128 - 1) // (16 * 128)) * (16 * 128)
    mesh = plsc.VectorSubcoreMesh(core_axis_name="c", subcore_axis_name="s")
    kern = pl.kernel(
        functools.partial(_edge_body, n_pad, ept, e),
        out_type=[
            jax.ShapeDtypeStruct((H, n_pad, C), jnp.float32),
            jax.ShapeDtypeStruct((2, n_pad, C), jnp.float32),
            jax.ShapeDtypeStruct((H * e,), jnp.float32),
        ],
        mesh=mesh,
        scratch_types=[
            pltpu.VMEM((NB,), jnp.int32),
            pltpu.VMEM((NB,), jnp.int32),
            pltpu.VMEM((NB,), jnp.int32),
            pltpu.VMEM((NB, QXW), jnp.float32),
            pltpu.VMEM((NB, C), jnp.float32),
            pltpu.VMEM((NB, C), jnp.float32),
            pltpu.VMEM((NB, ED), jnp.float32),
            pltpu.VMEM((NB,), jnp.float32),
            pltpu.VMEM((NB,), jnp.float32),
            pltpu.VMEM((NB, C), jnp.float32),
            pltpu.VMEM((32, C), jnp.float32),
            pltpu.VMEM_SHARED((n_pad, C), jnp.float32),
            pltpu.SemaphoreType.DMA,
        ],
    )
    heads = jnp.arange(H, dtype=jnp.int32)[:, None]
    qi = (dst[None, :] * H + heads).reshape(-1)
    ki = (src[None, :] * H + heads).reshape(-1)
    return kern(qx.reshape(n * H, QXW), k.reshape(n * H, C),
                v.reshape(n * H, C), ea, qi, ki, dst)


def _combine_body(num_ref, aux_ref, skip_ref, we_ref, g_ref, b_ref, wo_ref,
                  bo_ref, o_ref, *, relu, proj):
    we = we_ref[...]
    tot = None
    for h in range(H):
        hl = h % 2
        s = aux_ref[h // 2, :, hl * 32:hl * 32 + ED]
        den = aux_ref[h // 2, :, hl * 32 + ED:hl * 32 + ED + 1]
        numh = num_ref[h] + lax.dot_general(
            s, we[h * C:(h + 1) * C, :], (((1,), (1,)), ((), ())),
            preferred_element_type=jnp.float32)
        outh = numh / (den + 1e-16)
        tot = outh if tot is None else tot + outh
    m = tot * (1.0 / H) + skip_ref[...]
    mu = jnp.mean(m, axis=-1, keepdims=True)
    var = jnp.mean((m - mu) ** 2, axis=-1, keepdims=True)
    y = (m - mu) / jnp.sqrt(var + 1e-5) * g_ref[...] + b_ref[...]
    if relu:
        y = jnp.maximum(y, 0.0)
    if proj:
        y = lax.dot_general(y, wo_ref[...], (((1,), (1,)), ((), ())),
                            preferred_element_type=jnp.float32) + bo_ref[...]
    o_ref[...] = y


def _combine(num, aux, skip, we, g, b, wo, bo, relu, proj):
    n = skip.shape[0]
    bn = 1000
    out_w = wo.shape[0] if proj else C
    full = lambda s: pl.BlockSpec(s, lambda i: (0,) * len(s))
    return pl.pallas_call(
        functools.partial(_combine_body, relu=relu, proj=proj),
        grid=(n // bn,),
        in_specs=[pl.BlockSpec((H, bn, C), lambda i: (0, i, 0)),
                  pl.BlockSpec((2, bn, C), lambda i: (0, i, 0)),
                  pl.BlockSpec((bn, C), lambda i: (i, 0)),
                  full(we.shape), full((1, C)), full((1, C)),
                  full(wo.shape), full((1, wo.shape[0]))],
        out_specs=pl.BlockSpec((bn, out_w), lambda i: (i, 0)),
        out_shape=jax.ShapeDtypeStruct((n, out_w), jnp.float32),
    )(num[:, :n], aux[:, :n], skip, we, g.reshape(1, -1), b.reshape(1, -1),
      wo, bo.reshape(1, -1))


def kernel(x, edge_index, edge_attr, params):
    src = edge_index[0].astype(jnp.int32)
    dst = edge_index[1].astype(jnp.int32)
    g, b = params["ln_g"], params["ln_b"]

    p = params["c1"]
    qx, k, v, skip = _pre(x, p)
    num, aux, _ = _edge_pass(qx, k, v, edge_attr, src, dst)
    h1 = _combine(num, aux, skip, p["We"], g, b, params["Wout"],
                  params["bout"], relu=True, proj=False)

    p = params["c2"]
    qx, k, v, skip = _pre(h1, p)
    num, aux, _ = _edge_pass(qx, k, v, edge_attr, src, dst)
    out = _combine(num, aux, skip, p["We"], g, b, params["Wout"],
                   params["bout"], relu=False, proj=True)
    return out


# speedup vs baseline: 10.5321x; 1.5336x over previous
import functools

import jax
import jax.numpy as jnp
from jax import lax
from jax.experimental import pallas as pl
from jax.experimental.pallas import tpu as pltpu
from jax.experimental.pallas import tpu_sc as plsc

H = 4
C = 128
ED = 16
QXW = 256
NB = 32
INV_SQRT_C = 1.0 / float(C) ** 0.5


def _pre_body(x_ref, wq_ref, wk_ref, wv_ref, we_ref, ws_ref, bq_ref, bk_ref,
              bv_ref, bs_ref, qx_ref, k_ref, v_ref, skip_ref):
    xb = x_ref[...]
    dn = (((1,), (1,)), ((), ()))
    q = lax.dot_general(xb, wq_ref[...], dn,
                        preferred_element_type=jnp.float32) + bq_ref[...]
    k_ref[...] = lax.dot_general(xb, wk_ref[...], dn,
                                 preferred_element_type=jnp.float32) + bk_ref[...]
    v_ref[...] = lax.dot_general(xb, wv_ref[...], dn,
                                 preferred_element_type=jnp.float32) + bv_ref[...]
    skip_ref[...] = lax.dot_general(
        xb, ws_ref[...], dn, preferred_element_type=jnp.float32) + bs_ref[...]
    we = we_ref[...]
    zpad = jnp.zeros((q.shape[0], QXW - C - ED), jnp.float32)
    for h in range(H):
        qh = q[:, h * C:(h + 1) * C]
        qe = lax.dot_general(qh, we[h * C:(h + 1) * C, :],
                             (((1,), (0,)), ((), ())),
                             preferred_element_type=jnp.float32)
        qx_ref[:, h * QXW:h * QXW + C] = qh
        qx_ref[:, h * QXW + C:h * QXW + C + ED] = qe
        qx_ref[:, h * QXW + C + ED:(h + 1) * QXW] = zpad


def _pre(x, p):
    n = x.shape[0]
    bn = 1000
    full = lambda s: pl.BlockSpec(s, lambda i: (0,) * len(s))
    row = lambda w: pl.BlockSpec((bn, w), lambda i: (i, 0))
    return pl.pallas_call(
        _pre_body,
        grid=(n // bn,),
        in_specs=[row(x.shape[1])] + [full(p[k].shape) for k in
                                      ("Wq", "Wk", "Wv", "We", "Wskip")]
        + [full((1, H * C))] * 3 + [full((1, C))],
        out_specs=[row(H * QXW), row(H * C), row(H * C), row(C)],
        out_shape=[
            jax.ShapeDtypeStruct((n, H * QXW), jnp.float32),
            jax.ShapeDtypeStruct((n, H * C), jnp.float32),
            jax.ShapeDtypeStruct((n, H * C), jnp.float32),
            jax.ShapeDtypeStruct((n, C), jnp.float32),
        ],
    )(x, p["Wq"], p["Wk"], p["Wv"], p["We"], p["Wskip"],
      p["bq"].reshape(1, -1), p["bk"].reshape(1, -1), p["bv"].reshape(1, -1),
      p["bskip"].reshape(1, -1))


def _edge_body(n_pad, ept, e_all, qx_hbm, k_hbm, v_hbm, ea_hbm, qk_hbm,
               dst_hbm, num_hbm, aux_hbm, ex_hbm, qkb, dstb, qxrows, krows,
               earows, exbuf, ex1buf, obuf, zbuf, shared,
               sem_gat, sem_idx, sem_ex, sem_sc):
    core = lax.axis_index("c")
    sub = lax.axis_index("s")
    ebase = sub * ept
    npt = n_pad // 16
    nbase = sub * npt
    nbatch = ept // NB
    nbg_all = e_all // NB
    gb0 = sub * nbatch
    zr = zbuf.shape[0]
    zeros16 = jnp.zeros((16,), jnp.float32)
    lane = lax.broadcasted_iota(jnp.int32, (16,), 0)
    onehot = jnp.where(lane == 0, 1.0, 0.0).astype(jnp.float32)
    gd = lax.GatherDimensionNumbers(
        offset_dims=(), collapsed_slice_dims=(0,), start_index_map=(0,))

    def lanesum(v):
        for sh in (8, 4, 2, 1):
            p = lax.gather(v, (lane ^ sh)[:, None], gd, slice_sizes=(1,),
                           mode=lax.GatherScatterMode.PROMISE_IN_BOUNDS)
            v = v + p
        return v

    def zero_zbuf(i, carry):
        for j in range(zbuf.shape[1] // 16):
            zbuf[i, pl.ds(j * 16, 16)] = zeros16
        return carry
    lax.fori_loop(0, zr, zero_zbuf, 0)

    def zero_shared():
        for z in range(npt // zr):
            pltpu.sync_copy(zbuf, shared.at[pl.ds(nbase + z * zr, zr)])

    def idx_cp(h, j):
        par = lax.rem(j, 2)
        p3 = lax.rem(j, 3)
        return (pltpu.make_async_copy(
                    qk_hbm.at[h * nbg_all + gb0 + j], qkb.at[par], sem_idx),
                pltpu.make_async_copy(
                    dst_hbm.at[pl.ds(ebase + j * NB, NB)], dstb.at[p3],
                    sem_idx))

    def gat_cp(j):
        par = lax.rem(j, 2)
        p3 = lax.rem(j, 3)
        return (pltpu.make_async_copy(qx_hbm.at[qkb.at[par, 0]],
                                      qxrows.at[par], sem_gat),
                pltpu.make_async_copy(k_hbm.at[qkb.at[par, 1]],
                                     krows.at[par], sem_gat),
                pltpu.make_async_copy(v_hbm.at[qkb.at[par, 1]],
                                     obuf.at[p3], sem_gat),
                pltpu.make_async_copy(ea_hbm.at[pl.ds(ebase + j * NB, NB)],
                                      earows.at[par], sem_gat))

    def ex_cp(h, j):
        par = lax.rem(j, 2)
        return pltpu.make_async_copy(
            exbuf.at[par],
            ex_hbm.at[pl.ds(h * e_all + ebase + j * NB, NB)], sem_ex)

    def sc_cp(j):
        p3 = lax.rem(j, 3)
        return pltpu.make_async_copy(obuf.at[p3], shared.at[dstb.at[p3]],
                                     sem_sc)

    def start(cps):
        for cp in (cps if isinstance(cps, tuple) else (cps,)):
            cp.start()

    def wait(cps):
        for cp in (cps if isinstance(cps, tuple) else (cps,)):
            cp.wait()

    for hl in range(2):
        h = core * 2 + hl
        zero_shared()
        plsc.subcore_barrier()

        start(idx_cp(h, 0))
        start(idx_cp(h, 1))
        wait(idx_cp(h, 0))
        start(gat_cp(0))

        def batch_body(b, carry):
            par = lax.rem(b, 2)
            p3 = lax.rem(b, 3)
            wait(gat_cp(b))

            @pl.when(b + 1 < nbatch)
            def _():
                wait(idx_cp(h, b + 1))
                start(gat_cp(b + 1))

            @pl.when(b >= 1)
            def _():
                wait(ex_cp(h, b - 1))
                wait(sc_cp(b - 1))

            @pl.when(b + 2 < nbatch)
            def _():
                start(idx_cp(h, b + 2))

            def alpha_body(ii, avec):
                acc = (qxrows[par, ii, pl.ds(0, 16)]
                       * krows[par, ii, pl.ds(0, 16)])
                for j in range(1, C // 16):
                    acc += (qxrows[par, ii, pl.ds(j * 16, 16)]
                            * krows[par, ii, pl.ds(j * 16, 16)])
                acc = acc + (qxrows[par, ii, pl.ds(C, 16)]
                             * earows[par, ii, :])
                s = lanesum(acc)
                i16 = lax.rem(ii, 16)
                avec = jnp.where(lane == i16, s, avec)

                @pl.when(i16 == 15)
                def _():
                    exbuf[par, pl.ds(ii - 15, 16)] = jnp.exp(
                        avec * INV_SQRT_C)
                return jnp.where(i16 == 15, zeros16, avec)
            lax.fori_loop(0, NB, alpha_body, zeros16)
            start(ex_cp(h, b))

            def scale_body(ii, c2):
                exvec = exbuf[par, pl.ds((ii // 16) * 16, 16)]
                i16 = lax.rem(ii, 16)
                exs = lax.gather(exvec, jnp.full((16, 1), i16, jnp.int32),
                                 gd, slice_sizes=(1,),
                                 mode=lax.GatherScatterMode.PROMISE_IN_BOUNDS)
                for j in range(C // 16):
                    obuf[p3, ii, pl.ds(j * 16, 16)] = (
                        obuf[p3, ii, pl.ds(j * 16, 16)] * exs)
                return c2
            lax.fori_loop(0, NB, scale_body, 0)
            sc_cp(b).start(add=True)
            return carry
        lax.fori_loop(0, nbatch, batch_body, 0)
        wait(ex_cp(h, nbatch - 1))
        wait(sc_cp(nbatch - 1))
        plsc.subcore_barrier()
        pltpu.sync_copy(shared.at[pl.ds(nbase, npt)],
                        num_hbm.at[h, pl.ds(nbase, npt)])
        plsc.subcore_barrier()

    zero_shared()

    def zero_obuf(i, carry):
        for par in range(3):
            for j in range(4, C // 16):
                obuf[par, i, pl.ds(j * 16, 16)] = zeros16
        return carry
    lax.fori_loop(0, NB, zero_obuf, 0)
    plsc.subcore_barrier()

    h0 = core * 2

    def aux_ld(j):
        par = lax.rem(j, 2)
        p3 = lax.rem(j, 3)
        return (pltpu.make_async_copy(
                    dst_hbm.at[pl.ds(ebase + j * NB, NB)], dstb.at[p3],
                    sem_idx),
                pltpu.make_async_copy(
                    ex_hbm.at[pl.ds(h0 * e_all + ebase + j * NB, NB)],
                    exbuf.at[par], sem_gat),
                pltpu.make_async_copy(
                    ex_hbm.at[pl.ds((h0 + 1) * e_all + ebase + j * NB, NB)],
                    ex1buf.at[par], sem_gat),
                pltpu.make_async_copy(ea_hbm.at[pl.ds(ebase + j * NB, NB)],
                                      earows.at[par], sem_gat))

    start(aux_ld(0))
    start(aux_ld(1))

    def aux_body(b, carry):
        par = lax.rem(b, 2)
        p3 = lax.rem(b, 3)
        wait(aux_ld(b))

        @pl.when(b >= 2)
        def _():
            wait(sc_cp(b - 2))

        def aux_scale(ii, c2):
            base16 = (ii // 16) * 16
            i16 = lax.rem(ii, 16)
            idx16 = jnp.full((16, 1), i16, jnp.int32)
            exvec0 = exbuf[par, pl.ds(base16, 16)]
            exvec1 = ex1buf[par, pl.ds(base16, 16)]
            ex0 = lax.gather(exvec0, idx16, gd, slice_sizes=(1,),
                             mode=lax.GatherScatterMode.PROMISE_IN_BOUNDS)
            ex1 = lax.gather(exvec1, idx16, gd, slice_sizes=(1,),
                             mode=lax.GatherScatterMode.PROMISE_IN_BOUNDS)
            obuf[p3, ii, pl.ds(0, 16)] = earows[par, ii, :] * ex0
            obuf[p3, ii, pl.ds(16, 16)] = onehot * ex0
            obuf[p3, ii, pl.ds(32, 16)] = earows[par, ii, :] * ex1
            obuf[p3, ii, pl.ds(48, 16)] = onehot * ex1
            return c2
        lax.fori_loop(0, NB, aux_scale, 0)
        sc_cp(b).start(add=True)

        @pl.when(b + 2 < nbatch)
        def _():
            start(aux_ld(b + 2))
        return carry
    lax.fori_loop(0, nbatch, aux_body, 0)
    wait(sc_cp(nbatch - 2))
    wait(sc_cp(nbatch - 1))
    plsc.subcore_barrier()
    pltpu.sync_copy(shared.at[pl.ds(nbase, npt)],
                    aux_hbm.at[core, pl.ds(nbase, npt)])


def _edge_pass(qx, k, v, ea, src, dst):
    n = k.shape[0]
    e = src.shape[0]
    ept = e // 16
    n_pad = ((n + 16 * 128 - 1) // (16 * 128)) * (16 * 128)
    mesh = plsc.VectorSubcoreMesh(core_axis_name="c", subcore_axis_name="s")
    kern = pl.kernel(
        functools.partial(_edge_body, n_pad, ept, e),
        out_type=[
            jax.ShapeDtypeStruct((H, n_pad, C), jnp.float32),
            jax.ShapeDtypeStruct((2, n_pad, C), jnp.float32),
            jax.ShapeDtypeStruct((H * e,), jnp.float32),
        ],
        mesh=mesh,
        scratch_types=[
            pltpu.VMEM((2, 2, NB), jnp.int32),
            pltpu.VMEM((3, NB), jnp.int32),
            pltpu.VMEM((2, NB, QXW), jnp.float32),
            pltpu.VMEM((2, NB, C), jnp.float32),
            pltpu.VMEM((2, NB, ED), jnp.float32),
            pltpu.VMEM((2, NB), jnp.float32),
            pltpu.VMEM((2, NB), jnp.float32),
            pltpu.VMEM((3, NB, C), jnp.float32),
            pltpu.VMEM((8, C), jnp.float32),
            pltpu.VMEM_SHARED((n_pad, C), jnp.float32),
            pltpu.SemaphoreType.DMA,
            pltpu.SemaphoreType.DMA,
            pltpu.SemaphoreType.DMA,
            pltpu.SemaphoreType.DMA,
        ],
    )
    nbg_all = e // NB
    heads = jnp.arange(H, dtype=jnp.int32)[:, None]
    qi = (dst[None, :] * H + heads).reshape(H, nbg_all, NB)
    ki = (src[None, :] * H + heads).reshape(H, nbg_all, NB)
    qk = jnp.stack([qi, ki], axis=2).reshape(H * nbg_all, 2, NB)
    return kern(qx.reshape(n * H, QXW), k.reshape(n * H, C),
                v.reshape(n * H, C), ea, qk, dst)


def _combine_body(num_ref, aux_ref, skip_ref, we_ref, g_ref, b_ref, wo_ref,
                  bo_ref, o_ref, *, relu, proj):
    we = we_ref[...]
    tot = None
    for h in range(H):
        hl = h % 2
        s = aux_ref[h // 2, :, hl * 32:hl * 32 + ED]
        den = aux_ref[h // 2, :, hl * 32 + ED:hl * 32 + ED + 1]
        numh = num_ref[h] + lax.dot_general(
            s, we[h * C:(h + 1) * C, :], (((1,), (1,)), ((), ())),
            preferred_element_type=jnp.float32)
        outh = numh / (den + 1e-16)
        tot = outh if tot is None else tot + outh
    m = tot * (1.0 / H) + skip_ref[...]
    mu = jnp.mean(m, axis=-1, keepdims=True)
    var = jnp.mean((m - mu) ** 2, axis=-1, keepdims=True)
    y = (m - mu) / jnp.sqrt(var + 1e-5) * g_ref[...] + b_ref[...]
    if relu:
        y = jnp.maximum(y, 0.0)
    if proj:
        y = lax.dot_general(y, wo_ref[...], (((1,), (1,)), ((), ())),
                            preferred_element_type=jnp.float32) + bo_ref[...]
    o_ref[...] = y


def _combine(num, aux, skip, we, g, b, wo, bo, relu, proj):
    n = skip.shape[0]
    bn = 1000
    out_w = wo.shape[0] if proj else C
    full = lambda s: pl.BlockSpec(s, lambda i: (0,) * len(s))
    return pl.pallas_call(
        functools.partial(_combine_body, relu=relu, proj=proj),
        grid=(n // bn,),
        in_specs=[pl.BlockSpec((H, bn, C), lambda i: (0, i, 0)),
                  pl.BlockSpec((2, bn, C), lambda i: (0, i, 0)),
                  pl.BlockSpec((bn, C), lambda i: (i, 0)),
                  full(we.shape), full((1, C)), full((1, C)),
                  full(wo.shape), full((1, wo.shape[0]))],
        out_specs=pl.BlockSpec((bn, out_w), lambda i: (i, 0)),
        out_shape=jax.ShapeDtypeStruct((n, out_w), jnp.float32),
    )(num[:, :n], aux[:, :n], skip, we, g.reshape(1, -1), b.reshape(1, -1),
      wo, bo.reshape(1, -1))


def kernel(x, edge_index, edge_attr, params):
    src = edge_index[0].astype(jnp.int32)
    dst = edge_index[1].astype(jnp.int32)
    g, b = params["ln_g"], params["ln_b"]

    p = params["c1"]
    qx, k, v, skip = _pre(x, p)
    num, aux, _ = _edge_pass(qx, k, v, edge_attr, src, dst)
    h1 = _combine(num, aux, skip, p["We"], g, b, params["Wout"],
                  params["bout"], relu=True, proj=False)

    p = params["c2"]
    qx, k, v, skip = _pre(h1, p)
    num, aux, _ = _edge_pass(qx, k, v, edge_attr, src, dst)
    out = _combine(num, aux, skip, p["We"], g, b, params["Wout"],
                   params["bout"], relu=False, proj=True)
    return out
